# baseline (speedup 1.0000x reference)
import functools

import jax
import jax.numpy as jnp
from jax import lax
from jax.experimental import pallas as pl
from jax.experimental.pallas import tpu as pltpu, tpu_sc as plsc


def _gather_kernel(B, V, D, NC, NS):
    NW = NC * NS
    b_per_w = B // NW
    mesh = plsc.VectorSubcoreMesh(core_axis_name="c", subcore_axis_name="s")

    @functools.partial(
        pl.kernel,
        mesh=mesh,
        out_type=jax.ShapeDtypeStruct((B, D), jnp.float32),
        scratch_types=[
            pltpu.VMEM((b_per_w,), jnp.int32),
            pltpu.VMEM((b_per_w, D), jnp.float32),
            pltpu.SemaphoreType.DMA,
            pltpu.SemaphoreType.DMA,
        ],
    )
    def k(nids_hbm, mem_hbm, out_hbm, idx_v, rows_v, gsem, wsem):
        wid = lax.axis_index("s") * NC + lax.axis_index("c")
        base = wid * b_per_w
        pltpu.sync_copy(nids_hbm.at[pl.ds(base, b_per_w)], idx_v)
        g = pltpu.async_copy(mem_hbm.at[idx_v], rows_v, gsem)
        w = pltpu.async_copy(rows_v, out_hbm.at[pl.ds(base, b_per_w)], wsem)
        w.wait()
        g.wait()

    return k


def kernel(nids, memory):
    (B,) = nids.shape
    V, D = memory.shape
    info = plsc.get_sparse_core_info()
    return _gather_kernel(B, V, D, info.num_cores, info.num_subcores)(
        nids, memory
    )


# speedup vs baseline: 1.0010x; 1.0010x over previous
import functools

import jax
import jax.numpy as jnp
from jax import lax
from jax.experimental import pallas as pl
from jax.experimental.pallas import tpu as pltpu, tpu_sc as plsc


def _gather_kernel(B, V, D, NC, NS):
    NW = NC * NS
    b_per_w = B // NW
    mesh = plsc.VectorSubcoreMesh(core_axis_name="c", subcore_axis_name="s")

    @functools.partial(
        pl.kernel,
        mesh=mesh,
        out_type=jax.ShapeDtypeStruct((B, D), jnp.float32),
        scratch_types=[
            pltpu.VMEM((b_per_w,), jnp.int32),
            pltpu.VMEM((b_per_w, D), jnp.float32),
            pltpu.SemaphoreType.DMA,
            pltpu.SemaphoreType.DMA,
        ],
    )
    def k(nids_hbm, mem_hbm, out_hbm, idx_v, rows_v, gsem, wsem):
        wid = lax.axis_index("s") * NC + lax.axis_index("c")
        base = wid * b_per_w
        pltpu.sync_copy(nids_hbm.at[pl.ds(base, b_per_w)], idx_v)
        pltpu.async_copy(mem_hbm.at[idx_v], rows_v, gsem).wait()
        pltpu.async_copy(rows_v, out_hbm.at[pl.ds(base, b_per_w)], wsem).wait()

    return k


def kernel(nids, memory):
    (B,) = nids.shape
    V, D = memory.shape
    info = plsc.get_sparse_core_info()
    return _gather_kernel(B, V, D, info.num_cores, info.num_subcores)(
        nids, memory
    )
